# baseline (speedup 1.0000x reference)
import functools

import jax
import jax.numpy as jnp
from jax import lax
from jax.experimental import pallas as pl
from jax.experimental.pallas import tpu as pltpu
from jax.experimental.pallas import tpu_sc as plsc

_N = 100000
_C = 16
_K = 27
_CH = 128
_CHK = _K * _CH
_NC = 2
_NS = 16
_NW = _NC * _NS
_CPW = 25
_NPAD = _NW * _CPW * _CH
_R = _NPAD * _K
_TROWS = _N + 8
_EPS = 1e-3
_BN = 1024
_GRID = _NPAD // _BN


def _mm_stats_body(g_ref, w_ref, o_ref, st_ref, acc_ref):
    i = pl.program_id(0)
    out = jnp.dot(g_ref[...], w_ref[...], preferred_element_type=jnp.float32)
    o_ref[...] = out

    @pl.when(i == 0)
    def _():
        acc_ref[...] = jnp.zeros((2, _C), jnp.float32)

    s = jnp.sum(out, axis=0, keepdims=True)
    q = jnp.sum(out * out, axis=0, keepdims=True)
    acc_ref[...] = acc_ref[...] + jnp.concatenate([s, q], axis=0)

    @pl.when(i == _GRID - 1)
    def _():
        st_ref[...] = acc_ref[...]


def _mm_stats(gth, wstk):
    return pl.pallas_call(
        _mm_stats_body,
        grid=(_GRID,),
        in_specs=[
            pl.BlockSpec((_BN, _K * _C), lambda i: (i, 0)),
            pl.BlockSpec((_K * _C, _C), lambda i: (0, 0)),
        ],
        out_specs=[
            pl.BlockSpec((_BN, _C), lambda i: (i, 0)),
            pl.BlockSpec((2, _C), lambda i: (0, 0)),
        ],
        out_shape=[
            jax.ShapeDtypeStruct((_NPAD, _C), jnp.float32),
            jax.ShapeDtypeStruct((2, _C), jnp.float32),
        ],
        scratch_shapes=[pltpu.VMEM((2, _C), jnp.float32)],
    )(gth, wstk)


def _affine_from_stats(st, g, b):
    m = st[0:1, :] / _N
    v = st[1:2, :] / _N - m * m
    inv = lax.rsqrt(v + _EPS)
    a = g * inv
    c = b - m * a
    return a, c


def _affine_relu_body(x_ref, st_ref, g_ref, b_ref, o_ref):
    a, c = _affine_from_stats(st_ref[...], g_ref[...], b_ref[...])
    h = jnp.maximum(x_ref[...] * a + c, 0.0)
    rows = pl.program_id(0) * _BN + lax.broadcasted_iota(jnp.int32, (_BN, 1), 0)
    h = jnp.where(rows < _N, h, 0.0)
    o_ref[...] = h.astype(jnp.bfloat16)


def _affine_relu(x, st, g, b):
    return pl.pallas_call(
        _affine_relu_body,
        grid=(_GRID,),
        in_specs=[
            pl.BlockSpec((_BN, _C), lambda i: (i, 0)),
            pl.BlockSpec((2, _C), lambda i: (0, 0)),
            pl.BlockSpec((1, _C), lambda i: (0, 0)),
            pl.BlockSpec((1, _C), lambda i: (0, 0)),
        ],
        out_specs=pl.BlockSpec((_BN, _C), lambda i: (i, 0)),
        out_shape=jax.ShapeDtypeStruct((_NPAD, _C), jnp.bfloat16),
    )(x, st, g, b)


def _cast_bf16_body(x_ref, o_ref):
    o_ref[...] = x_ref[...].astype(jnp.bfloat16)


def _cast_bf16(x):
    return pl.pallas_call(
        _cast_bf16_body,
        grid=(_GRID,),
        in_specs=[pl.BlockSpec((_BN, _C), lambda i: (i, 0))],
        out_specs=pl.BlockSpec((_BN, _C), lambda i: (i, 0)),
        out_shape=jax.ShapeDtypeStruct((_NPAD, _C), jnp.bfloat16),
    )(x)


def _final_body(x_ref, st_ref, g_ref, b_ref, f_ref, o_ref):
    a, c = _affine_from_stats(st_ref[...], g_ref[...], b_ref[...])
    o_ref[...] = jnp.maximum(x_ref[...] * a + c + f_ref[...], 0.0)


def _final(x, st, g, b, f):
    return pl.pallas_call(
        _final_body,
        grid=(_GRID,),
        in_specs=[
            pl.BlockSpec((_BN, _C), lambda i: (i, 0)),
            pl.BlockSpec((2, _C), lambda i: (0, 0)),
            pl.BlockSpec((1, _C), lambda i: (0, 0)),
            pl.BlockSpec((1, _C), lambda i: (0, 0)),
            pl.BlockSpec((_BN, _C), lambda i: (i, 0)),
        ],
        out_specs=pl.BlockSpec((_BN, _C), lambda i: (i, 0)),
        out_shape=jax.ShapeDtypeStruct((_NPAD, _C), jnp.float32),
    )(x, st, g, b, f)


def _sc_body(f_hbm, idxh, gth_hbm, f_sp, idx_v, gth_v, semg, semw):
    sid = lax.axis_index("s")
    cid = lax.axis_index("c")
    wid = sid * _NC + cid

    @pl.when(sid == 0)
    def _():
        pltpu.sync_copy(f_hbm.at[pl.ds(0, _TROWS)], f_sp)

    plsc.subcore_barrier()

    def body(c, carry):
        slot = lax.rem(c, 2)
        pslot = 1 - slot

        @pl.when(c >= 2)
        def _():
            pltpu.make_async_copy(
                gth_v.at[slot], gth_hbm.at[pl.ds(0, _CHK)], semw).wait()

        @pl.when(c < _CPW)
        def _():
            g = wid * _CPW + c
            pltpu.sync_copy(idxh.at[g], idx_v.at[slot])

            def fire(j, cc):
                pltpu.async_copy(
                    f_sp.at[idx_v.at[slot, j]],
                    gth_v.at[slot, pl.ds(j * _CH, _CH)], semg)
                return cc

            lax.fori_loop(0, _K, fire, 0)

        @pl.when(c >= 1)
        def _():
            pltpu.make_async_copy(
                f_hbm.at[pl.ds(0, _CHK)], gth_v.at[pslot], semg).wait()
            g = wid * _CPW + c - 1
            pltpu.async_copy(
                gth_v.at[pslot], gth_hbm.at[pl.ds(g * _CHK, _CHK)], semw)

        return carry

    lax.fori_loop(0, _CPW + 1, body, 0)
    pltpu.make_async_copy(gth_v.at[0], gth_hbm.at[pl.ds(0, _CHK)], semw).wait()


@functools.cache
def _sc_gather_kernel():
    return functools.partial(
        pl.kernel,
        out_type=jax.ShapeDtypeStruct((_R, _C), jnp.bfloat16),
        mesh=plsc.VectorSubcoreMesh(
            core_axis_name="c", subcore_axis_name="s",
            num_cores=_NC, num_subcores=_NS),
        scratch_types=[
            pltpu.VMEM_SHARED((_TROWS, _C), jnp.bfloat16),
            pltpu.VMEM((2, _K, _CH), jnp.int32),
            pltpu.VMEM((2, _CHK, _C), jnp.bfloat16),
            pltpu.SemaphoreType.DMA,
            pltpu.SemaphoreType.DMA,
        ],
        compiler_params=pltpu.CompilerParams(use_tc_tiling_on_sc=False),
    )(_sc_body)


def kernel(features, neighbor_idx, W1, W2, gamma1, beta1, gamma2, beta2):
    nbr = neighbor_idx.astype(jnp.int32)
    fidx = jnp.where(nbr >= 0, nbr, _N)
    fidx = jnp.pad(fidx, ((0, _NPAD - _N), (0, 0)), constant_values=_N)
    idx3d = fidx.reshape(-1, _K, _CH)

    f_pad = jnp.pad(features, ((0, _NPAD - _N), (0, 0)))
    w1s = W1.reshape(_K * _C, _C).astype(jnp.bfloat16)
    w2s = W2.reshape(_K * _C, _C).astype(jnp.bfloat16)
    g1v = gamma1.reshape(1, _C)
    b1v = beta1.reshape(1, _C)
    g2v = gamma2.reshape(1, _C)
    b2v = beta2.reshape(1, _C)

    f_bf = _cast_bf16(f_pad)
    gth1 = _sc_gather_kernel()(f_bf, idx3d).reshape(_NPAD, _K * _C)
    out1, st1 = _mm_stats(gth1, w1s)
    h = _affine_relu(out1, st1, g1v, b1v)
    gth2 = _sc_gather_kernel()(h, idx3d).reshape(_NPAD, _K * _C)
    out2, st2 = _mm_stats(gth2, w2s)
    y = _final(out2, st2, g2v, b2v, f_pad)
    return y[:_N]


# speedup vs baseline: 1.0230x; 1.0230x over previous
import functools

import jax
import jax.numpy as jnp
from jax import lax
from jax.experimental import pallas as pl
from jax.experimental.pallas import tpu as pltpu
from jax.experimental.pallas import tpu_sc as plsc

_N = 100000
_C = 16
_K = 27
_CH = 128
_CHK = _K * _CH
_NC = 2
_NS = 16
_NW = _NC * _NS
_CPW = 25
_NPAD = _NW * _CPW * _CH
_R = _NPAD * _K
_TROWS = _N + 8
_EPS = 1e-3
_BN = 1024
_GRID = _NPAD // _BN


def _mm_stats_body(g_ref, w_ref, o_ref, st_ref, acc_ref):
    i = pl.program_id(0)
    out = jnp.dot(g_ref[...], w_ref[...], preferred_element_type=jnp.float32)
    o_ref[...] = out

    @pl.when(i == 0)
    def _():
        acc_ref[...] = jnp.zeros((2, _C), jnp.float32)

    s = jnp.sum(out, axis=0, keepdims=True)
    q = jnp.sum(out * out, axis=0, keepdims=True)
    acc_ref[...] = acc_ref[...] + jnp.concatenate([s, q], axis=0)

    @pl.when(i == _GRID - 1)
    def _():
        st_ref[...] = acc_ref[...]


def _mm_stats(gth, wstk):
    return pl.pallas_call(
        _mm_stats_body,
        grid=(_GRID,),
        in_specs=[
            pl.BlockSpec((_BN, _K * _C), lambda i: (i, 0)),
            pl.BlockSpec((_K * _C, _C), lambda i: (0, 0)),
        ],
        out_specs=[
            pl.BlockSpec((_BN, _C), lambda i: (i, 0)),
            pl.BlockSpec((2, _C), lambda i: (0, 0)),
        ],
        out_shape=[
            jax.ShapeDtypeStruct((_NPAD, _C), jnp.float32),
            jax.ShapeDtypeStruct((2, _C), jnp.float32),
        ],
        scratch_shapes=[pltpu.VMEM((2, _C), jnp.float32)],
    )(gth, wstk)


def _affine_from_stats(st, g, b):
    m = st[0:1, :] / _N
    v = st[1:2, :] / _N - m * m
    inv = lax.rsqrt(v + _EPS)
    a = g * inv
    c = b - m * a
    return a, c


def _affine_relu_body(x_ref, st_ref, g_ref, b_ref, o_ref):
    a, c = _affine_from_stats(st_ref[...], g_ref[...], b_ref[...])
    h = jnp.maximum(x_ref[...] * a + c, 0.0)
    rows = pl.program_id(0) * _BN + lax.broadcasted_iota(jnp.int32, (_BN, 1), 0)
    h = jnp.where(rows < _N, h, 0.0)
    o_ref[...] = h.astype(jnp.bfloat16)


def _affine_relu(x, st, g, b):
    return pl.pallas_call(
        _affine_relu_body,
        grid=(_GRID,),
        in_specs=[
            pl.BlockSpec((_BN, _C), lambda i: (i, 0)),
            pl.BlockSpec((2, _C), lambda i: (0, 0)),
            pl.BlockSpec((1, _C), lambda i: (0, 0)),
            pl.BlockSpec((1, _C), lambda i: (0, 0)),
        ],
        out_specs=pl.BlockSpec((_BN, _C), lambda i: (i, 0)),
        out_shape=jax.ShapeDtypeStruct((_NPAD, _C), jnp.bfloat16),
    )(x, st, g, b)


def _final_body(x_ref, st_ref, g_ref, b_ref, f_ref, o_ref):
    a, c = _affine_from_stats(st_ref[...], g_ref[...], b_ref[...])
    o_ref[...] = jnp.maximum(x_ref[...] * a + c + f_ref[...], 0.0)


def _final(x, st, g, b, f):
    return pl.pallas_call(
        _final_body,
        grid=(_GRID,),
        in_specs=[
            pl.BlockSpec((_BN, _C), lambda i: (i, 0)),
            pl.BlockSpec((2, _C), lambda i: (0, 0)),
            pl.BlockSpec((1, _C), lambda i: (0, 0)),
            pl.BlockSpec((1, _C), lambda i: (0, 0)),
            pl.BlockSpec((_BN, _C), lambda i: (i, 0)),
        ],
        out_specs=pl.BlockSpec((_BN, _C), lambda i: (i, 0)),
        out_shape=jax.ShapeDtypeStruct((_NPAD, _C), jnp.float32),
    )(x, st, g, b, f)


def _sc_body(f_hbm, idxh, gth_hbm, f_sp, idx_v, gth_v, semg, semw):
    sid = lax.axis_index("s")
    cid = lax.axis_index("c")
    wid = sid * _NC + cid

    @pl.when(sid == 0)
    def _():
        pltpu.sync_copy(f_hbm.at[pl.ds(0, _TROWS)], f_sp)

    plsc.subcore_barrier()

    def body(c, carry):
        slot = lax.rem(c, 2)
        pslot = 1 - slot

        @pl.when(c >= 2)
        def _():
            pltpu.make_async_copy(
                gth_v.at[slot], gth_hbm.at[pl.ds(0, _CHK)], semw).wait()

        @pl.when(c < _CPW)
        def _():
            g = wid * _CPW + c
            pltpu.sync_copy(idxh.at[g], idx_v.at[slot])
            pltpu.async_copy(f_sp.at[idx_v.at[slot]], gth_v.at[slot], semg)

        @pl.when(c >= 1)
        def _():
            pltpu.make_async_copy(
                f_hbm.at[pl.ds(0, _CHK)], gth_v.at[pslot], semg).wait()
            g = wid * _CPW + c - 1
            pltpu.async_copy(
                gth_v.at[pslot], gth_hbm.at[pl.ds(g * _CHK, _CHK)], semw)

        return carry

    lax.fori_loop(0, _CPW + 1, body, 0)
    pltpu.make_async_copy(gth_v.at[0], gth_hbm.at[pl.ds(0, _CHK)], semw).wait()


@functools.cache
def _sc_gather_kernel():
    return functools.partial(
        pl.kernel,
        out_type=jax.ShapeDtypeStruct((_R, _C), jnp.bfloat16),
        mesh=plsc.VectorSubcoreMesh(
            core_axis_name="c", subcore_axis_name="s",
            num_cores=_NC, num_subcores=_NS),
        scratch_types=[
            pltpu.VMEM_SHARED((_TROWS, _C), jnp.bfloat16),
            pltpu.VMEM((2, _CHK), jnp.int32),
            pltpu.VMEM((2, _CHK, _C), jnp.bfloat16),
            pltpu.SemaphoreType.DMA,
            pltpu.SemaphoreType.DMA,
        ],
        compiler_params=pltpu.CompilerParams(use_tc_tiling_on_sc=False),
    )(_sc_body)


def kernel(features, neighbor_idx, W1, W2, gamma1, beta1, gamma2, beta2):
    nbr = neighbor_idx.astype(jnp.int32)
    fidx = jnp.where(nbr >= 0, nbr, _N)
    fidx = jnp.pad(fidx, ((0, _NPAD - _N), (0, 0)), constant_values=_N)
    idx2d = fidx.reshape(-1, _CHK)

    f_pad = jnp.pad(features, ((0, _NPAD - _N), (0, 0)))
    w1s = W1.reshape(_K * _C, _C).astype(jnp.bfloat16)
    w2s = W2.reshape(_K * _C, _C).astype(jnp.bfloat16)
    g1v = gamma1.reshape(1, _C)
    b1v = beta1.reshape(1, _C)
    g2v = gamma2.reshape(1, _C)
    b2v = beta2.reshape(1, _C)

    f_bf = f_pad.astype(jnp.bfloat16)
    gth1 = _sc_gather_kernel()(f_bf, idx2d).reshape(_NPAD, _K * _C)
    out1, st1 = _mm_stats(gth1, w1s)
    h = _affine_relu(out1, st1, g1v, b1v)
    gth2 = _sc_gather_kernel()(h, idx2d).reshape(_NPAD, _K * _C)
    out2, st2 = _mm_stats(gth2, w2s)
    y = _final(out2, st2, g2v, b2v, f_pad)
    return y[:_N]


# speedup vs baseline: 1.1603x; 1.1342x over previous
import functools

import jax
import jax.numpy as jnp
from jax import lax
from jax.experimental import pallas as pl
from jax.experimental.pallas import tpu as pltpu
from jax.experimental.pallas import tpu_sc as plsc

_N = 100000
_C = 16
_K = 27
_CH = 32
_CHK = _K * _CH
_NC = 2
_NS = 16
_NW = _NC * _NS
_CPW = 100
_NPAD = _NW * _CPW * _CH
_R = _NPAD * _K
_TROWS = _N + 8
_EPS = 1e-3
_BN = 1024
_GRID = _NPAD // _BN


def _mm_stats_body(g_ref, w_ref, o_ref, st_ref, acc_ref):
    i = pl.program_id(0)
    out = jnp.dot(g_ref[...], w_ref[...], preferred_element_type=jnp.float32)
    o_ref[...] = out

    @pl.when(i == 0)
    def _():
        acc_ref[...] = jnp.zeros((2, _C), jnp.float32)

    s = jnp.sum(out, axis=0, keepdims=True)
    q = jnp.sum(out * out, axis=0, keepdims=True)
    acc_ref[...] = acc_ref[...] + jnp.concatenate([s, q], axis=0)

    @pl.when(i == _GRID - 1)
    def _():
        st_ref[...] = acc_ref[...]


def _mm_stats(gth, wstk):
    return pl.pallas_call(
        _mm_stats_body,
        grid=(_GRID,),
        in_specs=[
            pl.BlockSpec((_BN, _K * _C), lambda i: (i, 0)),
            pl.BlockSpec((_K * _C, _C), lambda i: (0, 0)),
        ],
        out_specs=[
            pl.BlockSpec((_BN, _C), lambda i: (i, 0)),
            pl.BlockSpec((2, _C), lambda i: (0, 0)),
        ],
        out_shape=[
            jax.ShapeDtypeStruct((_NPAD, _C), jnp.float32),
            jax.ShapeDtypeStruct((2, _C), jnp.float32),
        ],
        scratch_shapes=[pltpu.VMEM((2, _C), jnp.float32)],
    )(gth, wstk)


def _affine_from_stats(st, g, b):
    m = st[0:1, :] / _N
    v = st[1:2, :] / _N - m * m
    inv = lax.rsqrt(v + _EPS)
    a = g * inv
    c = b - m * a
    return a, c


def _affine_relu_body(x_ref, st_ref, g_ref, b_ref, o_ref):
    a, c = _affine_from_stats(st_ref[...], g_ref[...], b_ref[...])
    h = jnp.maximum(x_ref[...] * a + c, 0.0)
    rows = pl.program_id(0) * _BN + lax.broadcasted_iota(jnp.int32, (_BN, 1), 0)
    h = jnp.where(rows < _N, h, 0.0)
    o_ref[...] = h


def _affine_relu(x, st, g, b):
    return pl.pallas_call(
        _affine_relu_body,
        grid=(_GRID,),
        in_specs=[
            pl.BlockSpec((_BN, _C), lambda i: (i, 0)),
            pl.BlockSpec((2, _C), lambda i: (0, 0)),
            pl.BlockSpec((1, _C), lambda i: (0, 0)),
            pl.BlockSpec((1, _C), lambda i: (0, 0)),
        ],
        out_specs=pl.BlockSpec((_BN, _C), lambda i: (i, 0)),
        out_shape=jax.ShapeDtypeStruct((_NPAD, _C), jnp.float32),
    )(x, st, g, b)


def _final_body(x_ref, st_ref, g_ref, b_ref, f_ref, o_ref):
    a, c = _affine_from_stats(st_ref[...], g_ref[...], b_ref[...])
    o_ref[...] = jnp.maximum(x_ref[...] * a + c + f_ref[...], 0.0)


def _final(x, st, g, b, f):
    return pl.pallas_call(
        _final_body,
        grid=(_GRID,),
        in_specs=[
            pl.BlockSpec((_BN, _C), lambda i: (i, 0)),
            pl.BlockSpec((2, _C), lambda i: (0, 0)),
            pl.BlockSpec((1, _C), lambda i: (0, 0)),
            pl.BlockSpec((1, _C), lambda i: (0, 0)),
            pl.BlockSpec((_BN, _C), lambda i: (i, 0)),
        ],
        out_specs=pl.BlockSpec((_BN, _C), lambda i: (i, 0)),
        out_shape=jax.ShapeDtypeStruct((_NPAD, _C), jnp.float32),
    )(x, st, g, b, f)


def _sc_body(f_hbm, idxh, gth_hbm, f_sp, idx_v, gth_v, semg, semw):
    sid = lax.axis_index("s")
    cid = lax.axis_index("c")
    wid = sid * _NC + cid

    @pl.when(sid == 0)
    def _():
        pltpu.sync_copy(f_hbm.at[pl.ds(0, _TROWS)], f_sp)

    plsc.subcore_barrier()

    def body(c, carry):
        slot = lax.rem(c, 2)
        pslot = 1 - slot

        @pl.when(c >= 2)
        def _():
            pltpu.make_async_copy(
                gth_v.at[slot], gth_hbm.at[pl.ds(0, _CHK)], semw).wait()

        @pl.when(c < _CPW)
        def _():
            g = wid * _CPW + c
            pltpu.sync_copy(idxh.at[g], idx_v.at[slot])
            pltpu.async_copy(f_sp.at[idx_v.at[slot]], gth_v.at[slot], semg)

        @pl.when(c >= 1)
        def _():
            pltpu.make_async_copy(
                f_hbm.at[pl.ds(0, _CHK)], gth_v.at[pslot], semg).wait()
            g = wid * _CPW + c - 1
            pltpu.async_copy(
                gth_v.at[pslot], gth_hbm.at[pl.ds(g * _CHK, _CHK)], semw)

        return carry

    lax.fori_loop(0, _CPW + 1, body, 0)
    pltpu.make_async_copy(gth_v.at[0], gth_hbm.at[pl.ds(0, _CHK)], semw).wait()


@functools.cache
def _sc_gather_kernel():
    return functools.partial(
        pl.kernel,
        out_type=jax.ShapeDtypeStruct((_R, _C), jnp.float32),
        mesh=plsc.VectorSubcoreMesh(
            core_axis_name="c", subcore_axis_name="s",
            num_cores=_NC, num_subcores=_NS),
        scratch_types=[
            pltpu.VMEM_SHARED((_TROWS, _C), jnp.float32),
            pltpu.VMEM((2, _CHK), jnp.int32),
            pltpu.VMEM((2, _CHK, _C), jnp.float32),
            pltpu.SemaphoreType.DMA,
            pltpu.SemaphoreType.DMA,
        ],
        compiler_params=pltpu.CompilerParams(use_tc_tiling_on_sc=False),
    )(_sc_body)


def kernel(features, neighbor_idx, W1, W2, gamma1, beta1, gamma2, beta2):
    nbr = neighbor_idx.astype(jnp.int32)
    fidx = jnp.where(nbr >= 0, nbr, _N)
    fidx = jnp.pad(fidx, ((0, _NPAD - _N), (0, 0)), constant_values=_N)
    idx2d = fidx.reshape(-1, _CHK)

    f_pad = jnp.pad(features, ((0, _NPAD - _N), (0, 0)))
    w1s = W1.reshape(_K * _C, _C)
    w2s = W2.reshape(_K * _C, _C)
    g1v = gamma1.reshape(1, _C)
    b1v = beta1.reshape(1, _C)
    g2v = gamma2.reshape(1, _C)
    b2v = beta2.reshape(1, _C)

    gth1 = _sc_gather_kernel()(f_pad, idx2d).reshape(_NPAD, _K * _C)
    out1, st1 = _mm_stats(gth1, w1s)
    h = _affine_relu(out1, st1, g1v, b1v)
    gth2 = _sc_gather_kernel()(h, idx2d).reshape(_NPAD, _K * _C)
    out2, st2 = _mm_stats(gth2, w2s)
    y = _final(out2, st2, g2v, b2v, f_pad)
    return y[:_N]


# speedup vs baseline: 1.1810x; 1.0178x over previous
import functools

import jax
import jax.numpy as jnp
from jax import lax
from jax.experimental import pallas as pl
from jax.experimental.pallas import tpu as pltpu
from jax.experimental.pallas import tpu_sc as plsc

_N = 100000
_C = 16
_K = 27
_CH = 32
_CHK = _K * _CH
_NC = 2
_NS = 16
_NW = _NC * _NS
_CPW = 100
_NPAD = _NW * _CPW * _CH
_R = _NPAD * _K
_TROWS = _N + 8
_SLICE = _N // _NS
_EPS = 1e-3
_BN = 1024
_GRID = _NPAD // _BN
_FBN = 1000


def _mm_stats_body(g_ref, w_ref, gm_ref, bt_ref, o_ref, aff_ref, acc_ref):
    i = pl.program_id(0)
    out = jnp.dot(g_ref[...], w_ref[...], preferred_element_type=jnp.float32)
    o_ref[...] = out

    @pl.when(i == 0)
    def _():
        acc_ref[...] = jnp.zeros((2, _C), jnp.float32)

    s = jnp.sum(out, axis=0, keepdims=True)
    q = jnp.sum(out * out, axis=0, keepdims=True)
    acc_ref[...] = acc_ref[...] + jnp.concatenate([s, q], axis=0)

    @pl.when(i == _GRID - 1)
    def _():
        m = acc_ref[0:1, :] / _N
        v = acc_ref[1:2, :] / _N - m * m
        a = gm_ref[...] * lax.rsqrt(v + _EPS)
        c = bt_ref[...] - m * a
        aff_ref[...] = jnp.concatenate(
            [a, c, jnp.zeros((1, _C), jnp.float32)], axis=0)


def _mm_stats(gth, wstk, gm, bt):
    return pl.pallas_call(
        _mm_stats_body,
        grid=(_GRID,),
        in_specs=[
            pl.BlockSpec((_BN, _K * _C), lambda i: (i, 0)),
            pl.BlockSpec((_K * _C, _C), lambda i: (0, 0)),
            pl.BlockSpec((1, _C), lambda i: (0, 0)),
            pl.BlockSpec((1, _C), lambda i: (0, 0)),
        ],
        out_specs=[
            pl.BlockSpec((_BN, _C), lambda i: (i, 0)),
            pl.BlockSpec((3, _C), lambda i: (0, 0)),
        ],
        out_shape=[
            jax.ShapeDtypeStruct((_NPAD, _C), jnp.float32),
            jax.ShapeDtypeStruct((3, _C), jnp.float32),
        ],
        scratch_shapes=[pltpu.VMEM((2, _C), jnp.float32)],
    )(gth, wstk, gm, bt)


def _final_body(x_ref, aff_ref, f_ref, o_ref):
    a = aff_ref[0:1, :]
    c = aff_ref[1:2, :]
    o_ref[...] = jnp.maximum(x_ref[...] * a + c + f_ref[...], 0.0)


def _final(x, aff, f):
    return pl.pallas_call(
        _final_body,
        grid=(_N // _FBN,),
        in_specs=[
            pl.BlockSpec((_FBN, _C), lambda i: (i, 0)),
            pl.BlockSpec((3, _C), lambda i: (0, 0)),
            pl.BlockSpec((_FBN, _C), lambda i: (i, 0)),
        ],
        out_specs=pl.BlockSpec((_FBN, _C), lambda i: (i, 0)),
        out_shape=jax.ShapeDtypeStruct((_N, _C), jnp.float32),
    )(x, aff, f)


_PIECES = []
_off = 0
while _off < _SLICE:
    _ln = min(_CHK, _SLICE - _off)
    _PIECES.append((_off, _ln))
    _off += _ln


def _sc_body(src_hbm, aff_hbm, idxh, gth_hbm, f_sp, idx_v, gth_v, aff_v,
             semg, semw):
    sid = lax.axis_index("s")
    cid = lax.axis_index("c")
    wid = sid * _NC + cid

    pltpu.sync_copy(aff_hbm, aff_v)
    a = aff_v[0, :]
    c = aff_v[1, :]
    m = aff_v[2, :]
    base = sid * _SLICE
    for off, ln in _PIECES:
        pltpu.sync_copy(src_hbm.at[pl.ds(base + off, ln)],
                        gth_v.at[0, pl.ds(0, ln)])

        def xf(r, cc):
            x = gth_v[0, r, :]
            gth_v[0, r, :] = jnp.maximum(a * x + c, m * x)
            return cc

        lax.fori_loop(0, ln, xf, 0)
        pltpu.sync_copy(gth_v.at[0, pl.ds(0, ln)],
                        f_sp.at[pl.ds(base + off, ln)])

    @pl.when(sid == 0)
    def _():
        def zr(r, cc):
            gth_v[0, r, :] = jnp.zeros((_C,), jnp.float32)
            return cc

        lax.fori_loop(0, 8, zr, 0)
        pltpu.sync_copy(gth_v.at[0, pl.ds(0, 8)], f_sp.at[pl.ds(_N, 8)])

    plsc.subcore_barrier()

    def body(ch, carry):
        slot = lax.rem(ch, 2)
        pslot = 1 - slot

        @pl.when(ch >= 2)
        def _():
            pltpu.make_async_copy(
                gth_v.at[slot], gth_hbm.at[pl.ds(0, _CHK)], semw).wait()

        @pl.when(ch < _CPW)
        def _():
            g = wid * _CPW + ch
            pltpu.sync_copy(idxh.at[g], idx_v.at[slot])
            pltpu.async_copy(f_sp.at[idx_v.at[slot]], gth_v.at[slot], semg)

        @pl.when(ch >= 1)
        def _():
            pltpu.make_async_copy(
                src_hbm.at[pl.ds(0, _CHK)], gth_v.at[pslot], semg).wait()
            g = wid * _CPW + ch - 1
            pltpu.async_copy(
                gth_v.at[pslot], gth_hbm.at[pl.ds(g * _CHK, _CHK)], semw)

        return carry

    lax.fori_loop(0, _CPW + 1, body, 0)
    pltpu.make_async_copy(gth_v.at[0], gth_hbm.at[pl.ds(0, _CHK)], semw).wait()


@functools.cache
def _sc_gather_kernel():
    return functools.partial(
        pl.kernel,
        out_type=jax.ShapeDtypeStruct((_R, _C), jnp.float32),
        mesh=plsc.VectorSubcoreMesh(
            core_axis_name="c", subcore_axis_name="s",
            num_cores=_NC, num_subcores=_NS),
        scratch_types=[
            pltpu.VMEM_SHARED((_TROWS, _C), jnp.float32),
            pltpu.VMEM((2, _CHK), jnp.int32),
            pltpu.VMEM((2, _CHK, _C), jnp.float32),
            pltpu.VMEM((3, _C), jnp.float32),
            pltpu.SemaphoreType.DMA,
            pltpu.SemaphoreType.DMA,
        ],
        compiler_params=pltpu.CompilerParams(use_tc_tiling_on_sc=False),
    )(_sc_body)


def kernel(features, neighbor_idx, W1, W2, gamma1, beta1, gamma2, beta2):
    nbrf = neighbor_idx.reshape(-1).astype(jnp.int32)
    fidx = jnp.where(nbrf >= 0, nbrf, _N)
    fidx = jnp.pad(fidx, (0, (_NPAD - _N) * _K), constant_values=_N)
    idx2d = fidx.reshape(-1, _CHK)

    w1s = W1.reshape(_K * _C, _C)
    w2s = W2.reshape(_K * _C, _C)
    g1v = gamma1.reshape(1, _C)
    b1v = beta1.reshape(1, _C)
    g2v = gamma2.reshape(1, _C)
    b2v = beta2.reshape(1, _C)
    ident = jnp.concatenate(
        [jnp.ones((1, _C), jnp.float32), jnp.zeros((1, _C), jnp.float32),
         jnp.ones((1, _C), jnp.float32)], axis=0)

    gth1 = _sc_gather_kernel()(features, ident, idx2d).reshape(_NPAD, _K * _C)
    out1, aff1 = _mm_stats(gth1, w1s, g1v, b1v)
    gth2 = _sc_gather_kernel()(out1, aff1, idx2d).reshape(_NPAD, _K * _C)
    out2, aff2 = _mm_stats(gth2, w2s, g2v, b2v)
    return _final(out2, aff2, features)


# speedup vs baseline: 1.1908x; 1.0083x over previous
import functools

import jax
import jax.numpy as jnp
from jax import lax
from jax.experimental import pallas as pl
from jax.experimental.pallas import tpu as pltpu
from jax.experimental.pallas import tpu_sc as plsc

_N = 100000
_C = 16
_K = 27
_CH = 32
_CHK = _K * _CH
_NC = 2
_NS = 16
_NW = _NC * _NS
_CPW = 100
_NPAD = _NW * _CPW * _CH
_R = _NPAD * _K
_TROWS = _N + 8
_SLICE = _N // _NS
_REAL = _N * _K // _CHK
_NVR = _CHK // _C
_EPS = 1e-3
_BN = 1024
_GRID = _NPAD // _BN
_FBN = 1000


def _mm_stats_body(g_ref, w_ref, gm_ref, bt_ref, o_ref, aff_ref, acc_ref):
    i = pl.program_id(0)
    out = jnp.dot(g_ref[...], w_ref[...], preferred_element_type=jnp.float32)
    o_ref[...] = out

    @pl.when(i == 0)
    def _():
        acc_ref[...] = jnp.zeros((2, _C), jnp.float32)

    s = jnp.sum(out, axis=0, keepdims=True)
    q = jnp.sum(out * out, axis=0, keepdims=True)
    acc_ref[...] = acc_ref[...] + jnp.concatenate([s, q], axis=0)

    @pl.when(i == _GRID - 1)
    def _():
        m = acc_ref[0:1, :] / _N
        v = acc_ref[1:2, :] / _N - m * m
        a = gm_ref[...] * lax.rsqrt(v + _EPS)
        c = bt_ref[...] - m * a
        aff_ref[...] = jnp.concatenate(
            [a, c, jnp.zeros((1, _C), jnp.float32)], axis=0)


def _mm_stats(gth, wstk, gm, bt):
    return pl.pallas_call(
        _mm_stats_body,
        grid=(_GRID,),
        in_specs=[
            pl.BlockSpec((_BN, _K * _C), lambda i: (i, 0)),
            pl.BlockSpec((_K * _C, _C), lambda i: (0, 0)),
            pl.BlockSpec((1, _C), lambda i: (0, 0)),
            pl.BlockSpec((1, _C), lambda i: (0, 0)),
        ],
        out_specs=[
            pl.BlockSpec((_BN, _C), lambda i: (i, 0)),
            pl.BlockSpec((3, _C), lambda i: (0, 0)),
        ],
        out_shape=[
            jax.ShapeDtypeStruct((_NPAD, _C), jnp.float32),
            jax.ShapeDtypeStruct((3, _C), jnp.float32),
        ],
        scratch_shapes=[pltpu.VMEM((2, _C), jnp.float32)],
    )(gth, wstk, gm, bt)


def _final_body(x_ref, aff_ref, f_ref, o_ref):
    a = aff_ref[0:1, :]
    c = aff_ref[1:2, :]
    o_ref[...] = jnp.maximum(x_ref[...] * a + c + f_ref[...], 0.0)


def _final(x, aff, f):
    return pl.pallas_call(
        _final_body,
        grid=(_N // _FBN,),
        in_specs=[
            pl.BlockSpec((_FBN, _C), lambda i: (i, 0)),
            pl.BlockSpec((3, _C), lambda i: (0, 0)),
            pl.BlockSpec((_FBN, _C), lambda i: (i, 0)),
        ],
        out_specs=pl.BlockSpec((_FBN, _C), lambda i: (i, 0)),
        out_shape=jax.ShapeDtypeStruct((_N, _C), jnp.float32),
    )(x, aff, f)


_PIECES = []
_off = 0
while _off < _SLICE:
    _ln = min(_CHK, _SLICE - _off)
    _PIECES.append((_off, _ln))
    _off += _ln


def _sc_body(src_hbm, aff_hbm, idxh, gth_hbm, f_sp, idx_v, gth_v, aff_v,
             semg, semw):
    sid = lax.axis_index("s")
    cid = lax.axis_index("c")
    wid = sid * _NC + cid

    pltpu.sync_copy(aff_hbm, aff_v)
    a = aff_v[0, :]
    c = aff_v[1, :]
    m = aff_v[2, :]
    base = sid * _SLICE
    o0, l0 = _PIECES[0]
    pltpu.async_copy(src_hbm.at[pl.ds(base + o0, l0)],
                     gth_v.at[0, pl.ds(0, l0)], semg)
    for p, (off, ln) in enumerate(_PIECES):
        slot = p & 1
        pltpu.make_async_copy(src_hbm.at[pl.ds(0, ln)],
                              gth_v.at[slot, pl.ds(0, ln)], semg).wait()
        if p + 1 < len(_PIECES):
            if p >= 1:
                _, pln = _PIECES[p - 1]
                pltpu.make_async_copy(gth_v.at[1 - slot, pl.ds(0, pln)],
                                      f_sp.at[pl.ds(0, pln)], semw).wait()
            off2, ln2 = _PIECES[p + 1]
            pltpu.async_copy(src_hbm.at[pl.ds(base + off2, ln2)],
                             gth_v.at[1 - slot, pl.ds(0, ln2)], semg)

        def xf(r, cc):
            x = gth_v[slot, r, :]
            gth_v[slot, r, :] = jnp.maximum(a * x + c, m * x)
            return cc

        lax.fori_loop(0, ln, xf, 0)
        pltpu.async_copy(gth_v.at[slot, pl.ds(0, ln)],
                         f_sp.at[pl.ds(base + off, ln)], semw)
    for p in (len(_PIECES) - 2, len(_PIECES) - 1):
        _, pln = _PIECES[p]
        pltpu.make_async_copy(gth_v.at[p & 1, pl.ds(0, pln)],
                              f_sp.at[pl.ds(0, pln)], semw).wait()

    @pl.when(sid == 0)
    def _():
        def zr(r, cc):
            gth_v[0, r, :] = jnp.zeros((_C,), jnp.float32)
            return cc

        lax.fori_loop(0, 8, zr, 0)
        pltpu.sync_copy(gth_v.at[0, pl.ds(0, 8)], f_sp.at[pl.ds(_N, 8)])

    plsc.subcore_barrier()

    zrow = jnp.full((_C,), _N, jnp.int32)

    def body(ch, carry):
        slot = lax.rem(ch, 2)
        pslot = 1 - slot

        @pl.when(ch >= 2)
        def _():
            pltpu.make_async_copy(
                gth_v.at[slot], gth_hbm.at[pl.ds(0, _CHK)], semw).wait()

        @pl.when(ch < _CPW)
        def _():
            g = wid * _CPW + ch

            @pl.when(g < _REAL)
            def _():
                pltpu.sync_copy(idxh.at[pl.ds(g * _CHK, _CHK)], idx_v.at[slot])

                def msk(v, cc):
                    x = idx_v[slot, pl.ds(v * _C, _C)]
                    idx_v[slot, pl.ds(v * _C, _C)] = jnp.where(x < 0, zrow, x)
                    return cc

                lax.fori_loop(0, _NVR, msk, 0)

            @pl.when(g >= _REAL)
            def _():
                def fil(v, cc):
                    idx_v[slot, pl.ds(v * _C, _C)] = zrow
                    return cc

                lax.fori_loop(0, _NVR, fil, 0)

            pltpu.async_copy(f_sp.at[idx_v.at[slot]], gth_v.at[slot], semg)

        @pl.when(ch >= 1)
        def _():
            pltpu.make_async_copy(
                src_hbm.at[pl.ds(0, _CHK)], gth_v.at[pslot], semg).wait()
            g = wid * _CPW + ch - 1
            pltpu.async_copy(
                gth_v.at[pslot], gth_hbm.at[pl.ds(g * _CHK, _CHK)], semw)

        return carry

    lax.fori_loop(0, _CPW + 1, body, 0)
    pltpu.make_async_copy(gth_v.at[0], gth_hbm.at[pl.ds(0, _CHK)], semw).wait()


@functools.cache
def _sc_gather_kernel():
    return functools.partial(
        pl.kernel,
        out_type=jax.ShapeDtypeStruct((_R, _C), jnp.float32),
        mesh=plsc.VectorSubcoreMesh(
            core_axis_name="c", subcore_axis_name="s",
            num_cores=_NC, num_subcores=_NS),
        scratch_types=[
            pltpu.VMEM_SHARED((_TROWS, _C), jnp.float32),
            pltpu.VMEM((2, _CHK), jnp.int32),
            pltpu.VMEM((2, _CHK, _C), jnp.float32),
            pltpu.VMEM((3, _C), jnp.float32),
            pltpu.SemaphoreType.DMA,
            pltpu.SemaphoreType.DMA,
        ],
        compiler_params=pltpu.CompilerParams(use_tc_tiling_on_sc=False),
    )(_sc_body)


def kernel(features, neighbor_idx, W1, W2, gamma1, beta1, gamma2, beta2):
    idx1d = neighbor_idx.reshape(-1).astype(jnp.int32)

    w1s = W1.reshape(_K * _C, _C)
    w2s = W2.reshape(_K * _C, _C)
    g1v = gamma1.reshape(1, _C)
    b1v = beta1.reshape(1, _C)
    g2v = gamma2.reshape(1, _C)
    b2v = beta2.reshape(1, _C)
    ident = jnp.concatenate(
        [jnp.ones((1, _C), jnp.float32), jnp.zeros((1, _C), jnp.float32),
         jnp.ones((1, _C), jnp.float32)], axis=0)

    gth1 = _sc_gather_kernel()(features, ident, idx1d).reshape(_NPAD, _K * _C)
    out1, aff1 = _mm_stats(gth1, w1s, g1v, b1v)
    gth2 = _sc_gather_kernel()(out1, aff1, idx1d).reshape(_NPAD, _K * _C)
    out2, aff2 = _mm_stats(gth2, w2s, g2v, b2v)
    return _final(out2, aff2, features)


# speedup vs baseline: 1.2068x; 1.0134x over previous
import functools

import jax
import jax.numpy as jnp
from jax import lax
from jax.experimental import pallas as pl
from jax.experimental.pallas import tpu as pltpu
from jax.experimental.pallas import tpu_sc as plsc

_N = 100000
_C = 16
_K = 27
_CH = 32
_CHK = _K * _CH
_NC = 2
_NS = 16
_NW = _NC * _NS
_CPW = 100
_CPWH = 50
_NPAD = _NW * _CPW * _CH
_R = _NPAD * _K
_TROWS = _N + 8
_SLICE = _N // _NS
_REAL = _N * _K // _CHK
_NVR = _CHK // _C
_EPS = 1e-3
_BN = 1024
_GRID = _NPAD // _BN
_GRIDH = _GRID // 2
_NH = _NPAD // 2
_RH = _R // 2
_FBN = 1000


def _mm_stats_body(g_ref, w_ref, gm_ref, bt_ref, st_in_ref, o_ref,
                   st_ref, acc_ref, *, emit_affine):
    i = pl.program_id(0)
    out = jnp.dot(g_ref[...], w_ref[...], preferred_element_type=jnp.float32)
    o_ref[...] = out

    @pl.when(i == 0)
    def _():
        acc_ref[...] = jnp.zeros((2, _C), jnp.float32)

    s = jnp.sum(out, axis=0, keepdims=True)
    q = jnp.sum(out * out, axis=0, keepdims=True)
    acc_ref[...] = acc_ref[...] + jnp.concatenate([s, q], axis=0)

    @pl.when(i == _GRIDH - 1)
    def _():
        if not emit_affine:
            st_ref[...] = acc_ref[...]
        else:
            tot = acc_ref[...] + st_in_ref[...]
            m = tot[0:1, :] / _N
            v = tot[1:2, :] / _N - m * m
            a = gm_ref[...] * lax.rsqrt(v + _EPS)
            c = bt_ref[...] - m * a
            st_ref[...] = jnp.concatenate(
                [a, c, jnp.zeros((1, _C), jnp.float32)], axis=0)


def _mm_stats(gth, wstk, gm, bt, st_in, emit_affine):
    body = functools.partial(_mm_stats_body, emit_affine=emit_affine)
    return pl.pallas_call(
        body,
        grid=(_GRIDH,),
        in_specs=[
            pl.BlockSpec((_BN, _K * _C), lambda i: (i, 0)),
            pl.BlockSpec((_K * _C, _C), lambda i: (0, 0)),
            pl.BlockSpec((1, _C), lambda i: (0, 0)),
            pl.BlockSpec((1, _C), lambda i: (0, 0)),
            pl.BlockSpec((2, _C), lambda i: (0, 0)),
        ],
        out_specs=[
            pl.BlockSpec((_BN, _C), lambda i: (i, 0)),
            pl.BlockSpec((3 if emit_affine else 2, _C), lambda i: (0, 0)),
        ],
        out_shape=[
            jax.ShapeDtypeStruct((_NH, _C), jnp.float32),
            jax.ShapeDtypeStruct((3 if emit_affine else 2, _C), jnp.float32),
        ],
        scratch_shapes=[pltpu.VMEM((2, _C), jnp.float32)],
    )(gth, wstk, gm, bt, st_in)


def _final_body(x_ref, aff_ref, f_ref, o_ref):
    a = aff_ref[0:1, :]
    c = aff_ref[1:2, :]
    o_ref[...] = jnp.maximum(x_ref[...] * a + c + f_ref[...], 0.0)


def _final(x, aff, f):
    return pl.pallas_call(
        _final_body,
        grid=(_N // _FBN,),
        in_specs=[
            pl.BlockSpec((_FBN, _C), lambda i: (i, 0)),
            pl.BlockSpec((3, _C), lambda i: (0, 0)),
            pl.BlockSpec((_FBN, _C), lambda i: (i, 0)),
        ],
        out_specs=pl.BlockSpec((_FBN, _C), lambda i: (i, 0)),
        out_shape=jax.ShapeDtypeStruct((_N, _C), jnp.float32),
    )(x, aff, f)


_PIECES = []
_off = 0
while _off < _SLICE:
    _ln = min(_CHK, _SLICE - _off)
    _PIECES.append((_off, _ln))
    _off += _ln


def _sc_body(src_hbm, aff_hbm, idxh, gth_hbm, f_sp, idx_v, gth_v, aff_v,
             semg, semw, *, half):
    sid = lax.axis_index("s")
    cid = lax.axis_index("c")
    wid = sid * _NC + cid

    pltpu.sync_copy(aff_hbm, aff_v)
    a = aff_v[0, :]
    c = aff_v[1, :]
    m = aff_v[2, :]
    base = sid * _SLICE
    o0, l0 = _PIECES[0]
    pltpu.async_copy(src_hbm.at[pl.ds(base + o0, l0)],
                     gth_v.at[0, pl.ds(0, l0)], semg)
    for p, (off, ln) in enumerate(_PIECES):
        slot = p & 1
        pltpu.make_async_copy(src_hbm.at[pl.ds(0, ln)],
                              gth_v.at[slot, pl.ds(0, ln)], semg).wait()
        if p + 1 < len(_PIECES):
            if p >= 1:
                _, pln = _PIECES[p - 1]
                pltpu.make_async_copy(gth_v.at[1 - slot, pl.ds(0, pln)],
                                      f_sp.at[pl.ds(0, pln)], semw).wait()
            off2, ln2 = _PIECES[p + 1]
            pltpu.async_copy(src_hbm.at[pl.ds(base + off2, ln2)],
                             gth_v.at[1 - slot, pl.ds(0, ln2)], semg)

        def xf(r, cc):
            x = gth_v[slot, r, :]
            gth_v[slot, r, :] = jnp.maximum(a * x + c, m * x)
            return cc

        lax.fori_loop(0, ln, xf, 0)
        pltpu.async_copy(gth_v.at[slot, pl.ds(0, ln)],
                         f_sp.at[pl.ds(base + off, ln)], semw)
    for p in (len(_PIECES) - 2, len(_PIECES) - 1):
        _, pln = _PIECES[p]
        pltpu.make_async_copy(gth_v.at[p & 1, pl.ds(0, pln)],
                              f_sp.at[pl.ds(0, pln)], semw).wait()

    @pl.when(sid == 0)
    def _():
        def zr(r, cc):
            gth_v[0, r, :] = jnp.zeros((_C,), jnp.float32)
            return cc

        lax.fori_loop(0, 8, zr, 0)
        pltpu.sync_copy(gth_v.at[0, pl.ds(0, 8)], f_sp.at[pl.ds(_N, 8)])

    plsc.subcore_barrier()

    zrow = jnp.full((_C,), _N, jnp.int32)

    def body(ch, carry):
        slot = lax.rem(ch, 2)
        pslot = 1 - slot

        @pl.when(ch >= 2)
        def _():
            pltpu.make_async_copy(
                gth_v.at[slot], gth_hbm.at[pl.ds(0, _CHK)], semw).wait()

        @pl.when(ch < _CPWH)
        def _():
            g = half * (_NW * _CPWH) + wid * _CPWH + ch

            @pl.when(g < _REAL)
            def _():
                pltpu.sync_copy(idxh.at[pl.ds(g * _CHK, _CHK)], idx_v.at[slot])

                def msk(v, cc):
                    x = idx_v[slot, pl.ds(v * _C, _C)]
                    idx_v[slot, pl.ds(v * _C, _C)] = jnp.where(x < 0, zrow, x)
                    return cc

                lax.fori_loop(0, _NVR, msk, 0)

            @pl.when(g >= _REAL)
            def _():
                def fil(v, cc):
                    idx_v[slot, pl.ds(v * _C, _C)] = zrow
                    return cc

                lax.fori_loop(0, _NVR, fil, 0)

            pltpu.async_copy(f_sp.at[idx_v.at[slot]], gth_v.at[slot], semg)

        @pl.when(ch >= 1)
        def _():
            pltpu.make_async_copy(
                src_hbm.at[pl.ds(0, _CHK)], gth_v.at[pslot], semg).wait()
            g = wid * _CPWH + ch - 1
            pltpu.async_copy(
                gth_v.at[pslot], gth_hbm.at[pl.ds(g * _CHK, _CHK)], semw)

        return carry

    lax.fori_loop(0, _CPWH + 1, body, 0)
    pltpu.make_async_copy(gth_v.at[0], gth_hbm.at[pl.ds(0, _CHK)], semw).wait()


@functools.cache
def _sc_gather_kernel(half):
    return functools.partial(
        pl.kernel,
        out_type=jax.ShapeDtypeStruct((_RH, _C), jnp.float32),
        mesh=plsc.VectorSubcoreMesh(
            core_axis_name="c", subcore_axis_name="s",
            num_cores=_NC, num_subcores=_NS),
        scratch_types=[
            pltpu.VMEM_SHARED((_TROWS, _C), jnp.float32),
            pltpu.VMEM((2, _CHK), jnp.int32),
            pltpu.VMEM((2, _CHK, _C), jnp.float32),
            pltpu.VMEM((3, _C), jnp.float32),
            pltpu.SemaphoreType.DMA,
            pltpu.SemaphoreType.DMA,
        ],
        compiler_params=pltpu.CompilerParams(use_tc_tiling_on_sc=False),
    )(functools.partial(_sc_body, half=half))


def kernel(features, neighbor_idx, W1, W2, gamma1, beta1, gamma2, beta2):
    idx1d = neighbor_idx.reshape(-1).astype(jnp.int32)

    w1s = W1.reshape(_K * _C, _C)
    w2s = W2.reshape(_K * _C, _C)
    g1v = gamma1.reshape(1, _C)
    b1v = beta1.reshape(1, _C)
    g2v = gamma2.reshape(1, _C)
    b2v = beta2.reshape(1, _C)
    ident = jnp.concatenate(
        [jnp.ones((1, _C), jnp.float32), jnp.zeros((1, _C), jnp.float32),
         jnp.ones((1, _C), jnp.float32)], axis=0)

    z2 = jnp.zeros((2, _C), jnp.float32)
    g1a = _sc_gather_kernel(0)(features, ident, idx1d).reshape(_NH, _K * _C)
    g1b = _sc_gather_kernel(1)(features, ident, idx1d).reshape(_NH, _K * _C)
    o1a, st1 = _mm_stats(g1a, w1s, g1v, b1v, z2, False)
    o1b, aff1 = _mm_stats(g1b, w1s, g1v, b1v, st1, True)
    out1 = jnp.concatenate([o1a, o1b], axis=0)
    g2a = _sc_gather_kernel(0)(out1, aff1, idx1d).reshape(_NH, _K * _C)
    g2b = _sc_gather_kernel(1)(out1, aff1, idx1d).reshape(_NH, _K * _C)
    o2a, st2 = _mm_stats(g2a, w2s, g2v, b2v, z2, False)
    o2b, aff2 = _mm_stats(g2b, w2s, g2v, b2v, st2, True)
    out2 = jnp.concatenate([o2a, o2b], axis=0)
    return _final(out2, aff2, features)


# speedup vs baseline: 1.2385x; 1.0263x over previous
import functools

import jax
import jax.numpy as jnp
from jax import lax
from jax.experimental import pallas as pl
from jax.experimental.pallas import tpu as pltpu
from jax.experimental.pallas import tpu_sc as plsc

_N = 100000
_C = 16
_K = 27
_CH = 32
_CHK = _K * _CH
_NC = 2
_NS = 16
_NW = _NC * _NS
_CPW = 100
_CPWH = 50
_NPAD = _NW * _CPW * _CH
_R = _NPAD * _K
_TROWS = _N + 8
_SLICE = _N // _NS
_REAL = _N * _K // _CHK
_NVR = _CHK // _C
_EPS = 1e-3
_BN = 2048
_GRID = _NPAD // _BN
_GRIDH = _GRID // 2
_NH = _NPAD // 2
_RH = _R // 2
_FBN = 1000


def _mm_stats_body(g_ref, w_ref, gm_ref, bt_ref, st_in_ref, o_ref,
                   st_ref, acc_ref, *, emit_affine):
    i = pl.program_id(0)
    out = jnp.dot(g_ref[...], w_ref[...], preferred_element_type=jnp.float32)
    o_ref[...] = out

    @pl.when(i == 0)
    def _():
        acc_ref[...] = jnp.zeros((2, _C), jnp.float32)

    s = jnp.sum(out, axis=0, keepdims=True)
    q = jnp.sum(out * out, axis=0, keepdims=True)
    acc_ref[...] = acc_ref[...] + jnp.concatenate([s, q], axis=0)

    @pl.when(i == _GRIDH - 1)
    def _():
        if not emit_affine:
            st_ref[...] = acc_ref[...]
        else:
            tot = acc_ref[...] + st_in_ref[...]
            m = tot[0:1, :] / _N
            v = tot[1:2, :] / _N - m * m
            a = gm_ref[...] * lax.rsqrt(v + _EPS)
            c = bt_ref[...] - m * a
            st_ref[...] = jnp.concatenate(
                [a, c, jnp.zeros((1, _C), jnp.float32)], axis=0)


def _mm_stats(gth, wstk, gm, bt, st_in, emit_affine):
    body = functools.partial(_mm_stats_body, emit_affine=emit_affine)
    return pl.pallas_call(
        body,
        grid=(_GRIDH,),
        in_specs=[
            pl.BlockSpec((_BN, _K * _C), lambda i: (i, 0)),
            pl.BlockSpec((_K * _C, _C), lambda i: (0, 0)),
            pl.BlockSpec((1, _C), lambda i: (0, 0)),
            pl.BlockSpec((1, _C), lambda i: (0, 0)),
            pl.BlockSpec((2, _C), lambda i: (0, 0)),
        ],
        out_specs=[
            pl.BlockSpec((_BN, _C), lambda i: (i, 0)),
            pl.BlockSpec((3 if emit_affine else 2, _C), lambda i: (0, 0)),
        ],
        out_shape=[
            jax.ShapeDtypeStruct((_NH, _C), jnp.float32),
            jax.ShapeDtypeStruct((3 if emit_affine else 2, _C), jnp.float32),
        ],
        scratch_shapes=[pltpu.VMEM((2, _C), jnp.float32)],
    )(gth, wstk, gm, bt, st_in)


def _final_body(x_ref, aff_ref, f_ref, o_ref):
    a = aff_ref[0:1, :]
    c = aff_ref[1:2, :]
    o_ref[...] = jnp.maximum(x_ref[...] * a + c + f_ref[...], 0.0)


def _final(x, aff, f):
    return pl.pallas_call(
        _final_body,
        grid=(_N // _FBN,),
        in_specs=[
            pl.BlockSpec((_FBN, _C), lambda i: (i, 0)),
            pl.BlockSpec((3, _C), lambda i: (0, 0)),
            pl.BlockSpec((_FBN, _C), lambda i: (i, 0)),
        ],
        out_specs=pl.BlockSpec((_FBN, _C), lambda i: (i, 0)),
        out_shape=jax.ShapeDtypeStruct((_N, _C), jnp.float32),
    )(x, aff, f)


_PIECES = []
_off = 0
while _off < _SLICE:
    _ln = min(_CHK, _SLICE - _off)
    _PIECES.append((_off, _ln))
    _off += _ln


def _sc_body(src_hbm, aff_hbm, idxh, gth_hbm, f_sp, idx_v, gth_v, aff_v,
             semg, semw, *, half):
    sid = lax.axis_index("s")
    cid = lax.axis_index("c")
    wid = sid * _NC + cid

    pltpu.sync_copy(aff_hbm, aff_v)
    a = aff_v[0, :]
    c = aff_v[1, :]
    m = aff_v[2, :]
    base = sid * _SLICE
    o0, l0 = _PIECES[0]
    pltpu.async_copy(src_hbm.at[pl.ds(base + o0, l0)],
                     gth_v.at[0, pl.ds(0, l0)], semg)
    for p, (off, ln) in enumerate(_PIECES):
        slot = p & 1
        pltpu.make_async_copy(src_hbm.at[pl.ds(0, ln)],
                              gth_v.at[slot, pl.ds(0, ln)], semg).wait()
        if p + 1 < len(_PIECES):
            if p >= 1:
                _, pln = _PIECES[p - 1]
                pltpu.make_async_copy(gth_v.at[1 - slot, pl.ds(0, pln)],
                                      f_sp.at[pl.ds(0, pln)], semw).wait()
            off2, ln2 = _PIECES[p + 1]
            pltpu.async_copy(src_hbm.at[pl.ds(base + off2, ln2)],
                             gth_v.at[1 - slot, pl.ds(0, ln2)], semg)

        def xf(r, cc):
            x = gth_v[slot, r, :]
            gth_v[slot, r, :] = jnp.maximum(a * x + c, m * x)
            return cc

        lax.fori_loop(0, ln, xf, 0)
        pltpu.async_copy(gth_v.at[slot, pl.ds(0, ln)],
                         f_sp.at[pl.ds(base + off, ln)], semw)
    for p in (len(_PIECES) - 2, len(_PIECES) - 1):
        _, pln = _PIECES[p]
        pltpu.make_async_copy(gth_v.at[p & 1, pl.ds(0, pln)],
                              f_sp.at[pl.ds(0, pln)], semw).wait()

    @pl.when(sid == 0)
    def _():
        def zr(r, cc):
            gth_v[0, r, :] = jnp.zeros((_C,), jnp.float32)
            return cc

        lax.fori_loop(0, 8, zr, 0)
        pltpu.sync_copy(gth_v.at[0, pl.ds(0, 8)], f_sp.at[pl.ds(_N, 8)])

    plsc.subcore_barrier()

    zrow = jnp.full((_C,), _N, jnp.int32)

    def body(ch, carry):
        slot = lax.rem(ch, 2)
        pslot = 1 - slot

        @pl.when(ch >= 2)
        def _():
            pltpu.make_async_copy(
                gth_v.at[slot], gth_hbm.at[pl.ds(0, _CHK)], semw).wait()

        @pl.when(ch < _CPWH)
        def _():
            g = half * (_NW * _CPWH) + wid * _CPWH + ch

            @pl.when(g < _REAL)
            def _():
                pltpu.sync_copy(idxh.at[pl.ds(g * _CHK, _CHK)], idx_v.at[slot])

                def msk(v, cc):
                    x = idx_v[slot, pl.ds(v * _C, _C)]
                    idx_v[slot, pl.ds(v * _C, _C)] = jnp.where(x < 0, zrow, x)
                    return cc

                lax.fori_loop(0, _NVR, msk, 0)

            @pl.when(g >= _REAL)
            def _():
                def fil(v, cc):
                    idx_v[slot, pl.ds(v * _C, _C)] = zrow
                    return cc

                lax.fori_loop(0, _NVR, fil, 0)

            pltpu.async_copy(f_sp.at[idx_v.at[slot]], gth_v.at[slot], semg)

        @pl.when(ch >= 1)
        def _():
            pltpu.make_async_copy(
                src_hbm.at[pl.ds(0, _CHK)], gth_v.at[pslot], semg).wait()
            g = wid * _CPWH + ch - 1
            pltpu.async_copy(
                gth_v.at[pslot], gth_hbm.at[pl.ds(g * _CHK, _CHK)], semw)

        return carry

    lax.fori_loop(0, _CPWH + 1, body, 0)
    pltpu.make_async_copy(gth_v.at[0], gth_hbm.at[pl.ds(0, _CHK)], semw).wait()


@functools.cache
def _sc_gather_kernel(half):
    return functools.partial(
        pl.kernel,
        out_type=jax.ShapeDtypeStruct((_RH, _C), jnp.float32),
        mesh=plsc.VectorSubcoreMesh(
            core_axis_name="c", subcore_axis_name="s",
            num_cores=_NC, num_subcores=_NS),
        scratch_types=[
            pltpu.VMEM_SHARED((_TROWS, _C), jnp.float32),
            pltpu.VMEM((2, _CHK), jnp.int32),
            pltpu.VMEM((2, _CHK, _C), jnp.float32),
            pltpu.VMEM((3, _C), jnp.float32),
            pltpu.SemaphoreType.DMA,
            pltpu.SemaphoreType.DMA,
        ],
        compiler_params=pltpu.CompilerParams(use_tc_tiling_on_sc=False),
    )(functools.partial(_sc_body, half=half))


def kernel(features, neighbor_idx, W1, W2, gamma1, beta1, gamma2, beta2):
    idx1d = neighbor_idx.reshape(-1).astype(jnp.int32)

    w1s = W1.reshape(_K * _C, _C)
    w2s = W2.reshape(_K * _C, _C)
    g1v = gamma1.reshape(1, _C)
    b1v = beta1.reshape(1, _C)
    g2v = gamma2.reshape(1, _C)
    b2v = beta2.reshape(1, _C)
    ident = jnp.concatenate(
        [jnp.ones((1, _C), jnp.float32), jnp.zeros((1, _C), jnp.float32),
         jnp.ones((1, _C), jnp.float32)], axis=0)

    z2 = jnp.zeros((2, _C), jnp.float32)
    g1a = _sc_gather_kernel(0)(features, ident, idx1d).reshape(_NH, _K * _C)
    g1b = _sc_gather_kernel(1)(features, ident, idx1d).reshape(_NH, _K * _C)
    o1a, st1 = _mm_stats(g1a, w1s, g1v, b1v, z2, False)
    o1b, aff1 = _mm_stats(g1b, w1s, g1v, b1v, st1, True)
    out1 = jnp.concatenate([o1a, o1b], axis=0)
    g2a = _sc_gather_kernel(0)(out1, aff1, idx1d).reshape(_NH, _K * _C)
    g2b = _sc_gather_kernel(1)(out1, aff1, idx1d).reshape(_NH, _K * _C)
    o2a, st2 = _mm_stats(g2a, w2s, g2v, b2v, z2, False)
    o2b, aff2 = _mm_stats(g2b, w2s, g2v, b2v, st2, True)
    out2 = jnp.concatenate([o2a, o2b], axis=0)
    return _final(out2, aff2, features)
